# Initial kernel scaffold; baseline (speedup 1.0000x reference)
import jax
import jax.numpy as jnp
from jax.experimental import pallas as pl


def kernel(queries, mem):
    raise NotImplementedError("write your pallas kernel here")


# speedup vs baseline: 1.9400x; 1.9400x over previous
import functools

import jax
import jax.numpy as jnp
from jax.experimental import pallas as pl

_P = 512
_M = 2
_DB = 256


_TILE = 112


def _vq_kernel(q_ref, mem_ref, emb_ref, idx0_ref, idx1_ref, loss_ref):
    q = q_ref[...]
    mem = mem_ref[...]
    rows = q.shape[0]
    loss_acc = jnp.zeros((), jnp.float32)
    idx_refs = (idx0_ref, idx1_ref)
    for i in range(_M):
        qi = q[:, i * _DB:(i + 1) * _DB]
        mi = mem[i * _P:(i + 1) * _P, :]
        scores = jax.lax.dot_general(
            qi, mi, (((1,), (1,)), ((), ())),
            precision=jax.lax.Precision.HIGHEST,
            preferred_element_type=jnp.float32)
        mnorm = jnp.sum(mi * mi, axis=1)
        dist = mnorm[None, :] - 2.0 * scores
        idx = jnp.argmin(dist, axis=1).astype(jnp.int32)
        onehot = (jax.lax.broadcasted_iota(jnp.int32, (rows, _P), 1)
                  == idx[:, None]).astype(jnp.float32)
        gathered = jax.lax.dot_general(
            onehot, mi, (((1,), (0,)), ((), ())),
            precision=jax.lax.Precision.HIGHEST,
            preferred_element_type=jnp.float32)
        emb = (gathered - qi) + qi
        emb_ref[:, i * _DB:(i + 1) * _DB] = emb
        idx_refs[i][...] = jnp.reshape(idx + i * _P, (1, 1, rows))
        loss_acc = loss_acc + jnp.sum((emb - qi) ** 2)

    step = pl.program_id(0)

    @pl.when(step == 0)
    def _init():
        loss_ref[...] = jnp.zeros((1, 1), jnp.float32)

    loss_ref[...] += jnp.reshape(loss_acc, (1, 1))


def kernel(queries, mem):
    B, N, D = queries.shape
    bn = B * N
    flat_q = queries.reshape(bn, D)
    grid = (bn // _TILE,)
    emb, idx0, idx1, loss = pl.pallas_call(
        _vq_kernel,
        grid=grid,
        in_specs=[
            pl.BlockSpec((_TILE, D), lambda i: (i, 0)),
            pl.BlockSpec((_M * _P, _DB), lambda i: (0, 0)),
        ],
        out_specs=(
            pl.BlockSpec((_TILE, D), lambda i: (i, 0)),
            pl.BlockSpec((1, 1, _TILE), lambda i: (i, 0, 0)),
            pl.BlockSpec((1, 1, _TILE), lambda i: (i, 0, 0)),
            pl.BlockSpec((1, 1), lambda i: (0, 0)),
        ),
        out_shape=(
            jax.ShapeDtypeStruct((bn, D), jnp.float32),
            jax.ShapeDtypeStruct((grid[0], 1, _TILE), jnp.int32),
            jax.ShapeDtypeStruct((grid[0], 1, _TILE), jnp.int32),
            jax.ShapeDtypeStruct((1, 1), jnp.float32),
        ),
    )(flat_q, mem)
    embeddings = emb.reshape(B, N, D)
    indices = jnp.stack(
        [idx0.reshape(bn), idx1.reshape(bn)], axis=1).reshape(B, N, _M)
    vq_loss = jnp.zeros((), jnp.float32)
    commitment_loss = (loss / jnp.float32(bn * D)).reshape(())
    return (embeddings, indices, vq_loss, commitment_loss)


# speedup vs baseline: 13.4511x; 6.9336x over previous
import jax
import jax.numpy as jnp
from jax.experimental import pallas as pl
from jax.experimental.pallas import tpu as pltpu

_P = 512
_M = 2
_DB = 256
_D = _M * _DB
_TILE = 112


def _vq_kernel(q_ref, wd_ref, gd_ref, emb_ref, idx0_ref, idx1_ref, loss_ref,
               mnorm_ref):
    step = pl.program_id(0)

    @pl.when(step == 0)
    def _init():
        wd = wd_ref[...]
        mnorm_ref[...] = jnp.sum(wd * wd, axis=0, keepdims=True)
        loss_ref[...] = jnp.zeros((1, 1), jnp.float32)

    q = q_ref[...]
    scores = jax.lax.dot_general(
        q, wd_ref[...], (((1,), (0,)), ((), ())),
        precision=jax.lax.Precision.HIGHEST,
        preferred_element_type=jnp.float32)
    dist = mnorm_ref[...] - 2.0 * scores

    rows = q.shape[0]
    iota = jax.lax.broadcasted_iota(jnp.int32, (rows, _P), 1)
    onehots = []
    idx_refs = (idx0_ref, idx1_ref)
    for i in range(_M):
        di = dist[:, i * _P:(i + 1) * _P]
        rmin = jnp.min(di, axis=1, keepdims=True)
        ismin = di == rmin
        idx = jnp.min(jnp.where(ismin, iota, _P),
                      axis=1, keepdims=True)
        onehots.append(jnp.logical_and(ismin, iota == idx))
        idx_refs[i][...] = idx + i * _P
    onehot = jnp.concatenate(onehots, axis=1).astype(jnp.float32)

    gathered = jax.lax.dot_general(
        onehot, gd_ref[...], (((1,), (0,)), ((), ())),
        precision=jax.lax.Precision.HIGHEST,
        preferred_element_type=jnp.float32)
    emb = (gathered - q) + q
    emb_ref[...] = emb
    loss_ref[...] += jnp.reshape(jnp.sum((emb - q) ** 2), (1, 1))


def kernel(queries, mem):
    B, N, D = queries.shape
    bn = B * N
    flat_q = queries.reshape(bn, D)
    memt = mem.T
    zc = jnp.zeros((_DB, _P), jnp.float32)
    wd = jnp.concatenate([
        jnp.concatenate([memt[:, :_P], zc], axis=1),
        jnp.concatenate([zc, memt[:, _P:]], axis=1),
    ], axis=0)
    zg = jnp.zeros((_P, _DB), jnp.float32)
    gd = jnp.concatenate([
        jnp.concatenate([mem[:_P], zg], axis=1),
        jnp.concatenate([zg, mem[_P:]], axis=1),
    ], axis=0)

    grid = (bn // _TILE,)
    emb, idx0, idx1, loss = pl.pallas_call(
        _vq_kernel,
        grid=grid,
        in_specs=[
            pl.BlockSpec((_TILE, D), lambda i: (i, 0)),
            pl.BlockSpec((_D, _M * _P), lambda i: (0, 0)),
            pl.BlockSpec((_M * _P, _D), lambda i: (0, 0)),
        ],
        out_specs=(
            pl.BlockSpec((_TILE, D), lambda i: (i, 0)),
            pl.BlockSpec((_TILE, 1), lambda i: (i, 0)),
            pl.BlockSpec((_TILE, 1), lambda i: (i, 0)),
            pl.BlockSpec((1, 1), lambda i: (0, 0)),
        ),
        out_shape=(
            jax.ShapeDtypeStruct((bn, D), jnp.float32),
            jax.ShapeDtypeStruct((bn, 1), jnp.int32),
            jax.ShapeDtypeStruct((bn, 1), jnp.int32),
            jax.ShapeDtypeStruct((1, 1), jnp.float32),
        ),
        scratch_shapes=[pltpu.VMEM((1, _M * _P), jnp.float32)],
    )(flat_q, wd, gd)
    embeddings = emb.reshape(B, N, D)
    indices = jnp.concatenate([idx0, idx1], axis=1).reshape(B, N, _M)
    vq_loss = jnp.zeros((), jnp.float32)
    commitment_loss = (loss / jnp.float32(bn * D)).reshape(())
    return (embeddings, indices, vq_loss, commitment_loss)


# speedup vs baseline: 17.6052x; 1.3088x over previous
import jax
import jax.numpy as jnp
from jax.experimental import pallas as pl
from jax.experimental.pallas import tpu as pltpu

_P = 512
_M = 2
_DB = 256
_D = _M * _DB
_TILE = 112


def _vq_kernel(q_ref, mem_ref, emb_ref, idx0_ref, idx1_ref, loss_ref,
               wdt_ref, mnorm_ref, mhi_ref, mlo_ref):
    step = pl.program_id(0)

    @pl.when(step == 0)
    def _init():
        m = mem_ref[...]
        wdt = jnp.swapaxes(m, 0, 1)
        wdt_ref[...] = wdt
        mnorm_ref[...] = jnp.sum(wdt * wdt, axis=0, keepdims=True)
        mhi = m.astype(jnp.bfloat16)
        mhi_ref[...] = mhi
        mlo_ref[...] = (m - mhi.astype(jnp.float32)).astype(jnp.bfloat16)
        loss_ref[...] = jnp.zeros((1, 1), jnp.float32)

    q = q_ref[...]
    rows = q.shape[0]
    iota = jax.lax.broadcasted_iota(jnp.int32, (rows, _P), 1)
    idx_refs = (idx0_ref, idx1_ref)
    loss_acc = jnp.zeros((), jnp.float32)
    for i in range(_M):
        qi = q[:, i * _DB:(i + 1) * _DB]
        scores = jax.lax.dot_general(
            qi, wdt_ref[:, i * _P:(i + 1) * _P],
            (((1,), (0,)), ((), ())),
            precision=jax.lax.Precision.HIGHEST,
            preferred_element_type=jnp.float32)
        dist = mnorm_ref[:, i * _P:(i + 1) * _P] - 2.0 * scores
        rmin = jnp.min(dist, axis=1, keepdims=True)
        ismin = dist == rmin
        idx = jnp.min(jnp.where(ismin, iota, _P),
                      axis=1, keepdims=True)
        idx_refs[i][...] = idx + i * _P
        onehot = jnp.logical_and(ismin, iota == idx).astype(jnp.bfloat16)
        gathered = jax.lax.dot_general(
            onehot, mhi_ref[i * _P:(i + 1) * _P, :],
            (((1,), (0,)), ((), ())),
            preferred_element_type=jnp.float32)
        gathered = gathered + jax.lax.dot_general(
            onehot, mlo_ref[i * _P:(i + 1) * _P, :],
            (((1,), (0,)), ((), ())),
            preferred_element_type=jnp.float32)
        emb = (gathered - qi) + qi
        emb_ref[:, i * _DB:(i + 1) * _DB] = emb
        loss_acc = loss_acc + jnp.sum((emb - qi) ** 2)
    loss_ref[...] += jnp.reshape(loss_acc, (1, 1))


def kernel(queries, mem):
    B, N, D = queries.shape
    bn = B * N
    flat_q = queries.reshape(bn, D)
    grid = (bn // _TILE,)
    emb, idx0, idx1, loss = pl.pallas_call(
        _vq_kernel,
        grid=grid,
        in_specs=[
            pl.BlockSpec((_TILE, D), lambda i: (i, 0)),
            pl.BlockSpec((_M * _P, _DB), lambda i: (0, 0)),
        ],
        out_specs=(
            pl.BlockSpec((_TILE, D), lambda i: (i, 0)),
            pl.BlockSpec((_TILE, 1), lambda i: (i, 0)),
            pl.BlockSpec((_TILE, 1), lambda i: (i, 0)),
            pl.BlockSpec((1, 1), lambda i: (0, 0)),
        ),
        out_shape=(
            jax.ShapeDtypeStruct((bn, D), jnp.float32),
            jax.ShapeDtypeStruct((bn, 1), jnp.int32),
            jax.ShapeDtypeStruct((bn, 1), jnp.int32),
            jax.ShapeDtypeStruct((1, 1), jnp.float32),
        ),
        scratch_shapes=[
            pltpu.VMEM((_DB, _M * _P), jnp.float32),
            pltpu.VMEM((1, _M * _P), jnp.float32),
            pltpu.VMEM((_M * _P, _DB), jnp.bfloat16),
            pltpu.VMEM((_M * _P, _DB), jnp.bfloat16),
        ],
    )(flat_q, mem)
    embeddings = emb.reshape(B, N, D)
    indices = jnp.concatenate([idx0, idx1], axis=1).reshape(B, N, _M)
    vq_loss = jnp.zeros((), jnp.float32)
    commitment_loss = (loss / jnp.float32(bn * D)).reshape(())
    return (embeddings, indices, vq_loss, commitment_loss)


# speedup vs baseline: 19.4498x; 1.1048x over previous
import functools

import jax
import jax.numpy as jnp
from jax.experimental import pallas as pl
from jax.experimental.pallas import tpu as pltpu

_P = 512
_M = 2
_DB = 256
_D = _M * _DB
_TILE = 112


def _vq_kernel(q_ref, mem_ref, emb_ref, idx_ref, loss_ref,
               wdt2_ref, mnorm_ref, mhi_ref, mlo_ref, *, inv_count):
    step = pl.program_id(0)
    nsteps = pl.num_programs(0)

    @pl.when(step == 0)
    def _init():
        m = mem_ref[...]
        wdt2 = jnp.swapaxes(m, 0, 1) * -2.0
        wdt2_ref[...] = wdt2
        mnorm_ref[...] = 0.25 * jnp.sum(wdt2 * wdt2, axis=0,
                                        keepdims=True)
        mhi = m.astype(jnp.bfloat16)
        mhi_ref[...] = mhi
        mlo_ref[...] = (m - mhi.astype(jnp.float32)).astype(jnp.bfloat16)
        loss_ref[...] = jnp.zeros((1, 1), jnp.float32)

    q = q_ref[...]
    rows = q.shape[0]
    iota = jax.lax.broadcasted_iota(jnp.int32, (rows, _P), 1)
    loss_acc = jnp.zeros((), jnp.float32)
    for i in range(_M):
        qi = q[:, i * _DB:(i + 1) * _DB]
        scores2 = jax.lax.dot_general(
            qi, wdt2_ref[:, i * _P:(i + 1) * _P],
            (((1,), (0,)), ((), ())),
            precision=jax.lax.Precision.HIGHEST,
            preferred_element_type=jnp.float32)
        dist = mnorm_ref[:, i * _P:(i + 1) * _P] + scores2
        rmin = jnp.min(dist, axis=1, keepdims=True)
        ismin = dist == rmin
        idx = jnp.min(jnp.where(ismin, iota, _P),
                      axis=1, keepdims=True)
        idx_ref[:, i:i + 1] = idx + i * _P
        onehot = jnp.logical_and(ismin, iota == idx).astype(jnp.bfloat16)
        gathered = jax.lax.dot_general(
            onehot, mhi_ref[i * _P:(i + 1) * _P, :],
            (((1,), (0,)), ((), ())),
            preferred_element_type=jnp.float32)
        gathered = gathered + jax.lax.dot_general(
            onehot, mlo_ref[i * _P:(i + 1) * _P, :],
            (((1,), (0,)), ((), ())),
            preferred_element_type=jnp.float32)
        emb = (gathered - qi) + qi
        emb_ref[:, i * _DB:(i + 1) * _DB] = emb
        loss_acc = loss_acc + jnp.sum((emb - qi) ** 2)
    loss_ref[...] += jnp.reshape(loss_acc, (1, 1))

    @pl.when(step == nsteps - 1)
    def _final():
        loss_ref[...] *= inv_count


def kernel(queries, mem):
    B, N, D = queries.shape
    bn = B * N
    flat_q = queries.reshape(bn, D)
    grid = (bn // _TILE,)
    emb, idx, loss = pl.pallas_call(
        functools.partial(_vq_kernel, inv_count=1.0 / float(bn * D)),
        grid=grid,
        in_specs=[
            pl.BlockSpec((_TILE, D), lambda i: (i, 0)),
            pl.BlockSpec((_M * _P, _DB), lambda i: (0, 0)),
        ],
        out_specs=(
            pl.BlockSpec((_TILE, D), lambda i: (i, 0)),
            pl.BlockSpec((_TILE, _M), lambda i: (i, 0)),
            pl.BlockSpec((1, 1), lambda i: (0, 0)),
        ),
        out_shape=(
            jax.ShapeDtypeStruct((bn, D), jnp.float32),
            jax.ShapeDtypeStruct((bn, _M), jnp.int32),
            jax.ShapeDtypeStruct((1, 1), jnp.float32),
        ),
        scratch_shapes=[
            pltpu.VMEM((_DB, _M * _P), jnp.float32),
            pltpu.VMEM((1, _M * _P), jnp.float32),
            pltpu.VMEM((_M * _P, _DB), jnp.bfloat16),
            pltpu.VMEM((_M * _P, _DB), jnp.bfloat16),
        ],
    )(flat_q, mem)
    embeddings = emb.reshape(B, N, D)
    indices = idx.reshape(B, N, _M)
    vq_loss = jnp.zeros((), jnp.float32)
    commitment_loss = loss.reshape(())
    return (embeddings, indices, vq_loss, commitment_loss)


# speedup vs baseline: 24.8368x; 1.2770x over previous
import functools

import jax
import jax.numpy as jnp
from jax.experimental import pallas as pl
from jax.experimental.pallas import tpu as pltpu

_P = 512
_M = 2
_DB = 256
_D = _M * _DB
_TILE = 392


def _vq_kernel(q_ref, mem_ref, emb_ref, idx_ref, loss_ref,
               wdt2_ref, mnorm_ref, mhi_ref, mlo_ref, *, inv_count):
    step = pl.program_id(0)
    nsteps = pl.num_programs(0)

    @pl.when(step == 0)
    def _init():
        m = mem_ref[...]
        wdt2 = jnp.swapaxes(m, 0, 1) * -2.0
        wdt2_ref[...] = wdt2
        mnorm_ref[...] = 0.25 * jnp.sum(wdt2 * wdt2, axis=0,
                                        keepdims=True)
        mhi = m.astype(jnp.bfloat16)
        mhi_ref[...] = mhi
        mlo_ref[...] = (m - mhi.astype(jnp.float32)).astype(jnp.bfloat16)
        loss_ref[...] = jnp.zeros((1, 1), jnp.float32)

    q = q_ref[...]
    rows = q.shape[0]
    iota = jax.lax.broadcasted_iota(jnp.int32, (rows, _P), 1)
    loss_acc = jnp.zeros((), jnp.float32)
    for i in range(_M):
        qi = q[:, i * _DB:(i + 1) * _DB]
        scores2 = jax.lax.dot_general(
            qi, wdt2_ref[:, i * _P:(i + 1) * _P],
            (((1,), (0,)), ((), ())),
            precision=jax.lax.Precision.HIGHEST,
            preferred_element_type=jnp.float32)
        dist = mnorm_ref[:, i * _P:(i + 1) * _P] + scores2
        rmin = jnp.min(dist, axis=1, keepdims=True)
        ismin = dist == rmin
        idx = jnp.min(jnp.where(ismin, iota, _P),
                      axis=1, keepdims=True)
        idx_ref[:, i:i + 1] = idx + i * _P
        onehot = jnp.logical_and(ismin, iota == idx).astype(jnp.bfloat16)
        gathered = jax.lax.dot_general(
            onehot, mhi_ref[i * _P:(i + 1) * _P, :],
            (((1,), (0,)), ((), ())),
            preferred_element_type=jnp.float32)
        gathered = gathered + jax.lax.dot_general(
            onehot, mlo_ref[i * _P:(i + 1) * _P, :],
            (((1,), (0,)), ((), ())),
            preferred_element_type=jnp.float32)
        emb = (gathered - qi) + qi
        emb_ref[:, i * _DB:(i + 1) * _DB] = emb
        loss_acc = loss_acc + jnp.sum((emb - qi) ** 2)
    loss_ref[...] += jnp.reshape(loss_acc, (1, 1))

    @pl.when(step == nsteps - 1)
    def _final():
        loss_ref[...] *= inv_count


def kernel(queries, mem):
    B, N, D = queries.shape
    bn = B * N
    flat_q = queries.reshape(bn, D)
    grid = (bn // _TILE,)
    emb, idx, loss = pl.pallas_call(
        functools.partial(_vq_kernel, inv_count=1.0 / float(bn * D)),
        grid=grid,
        in_specs=[
            pl.BlockSpec((_TILE, D), lambda i: (i, 0)),
            pl.BlockSpec((_M * _P, _DB), lambda i: (0, 0)),
        ],
        out_specs=(
            pl.BlockSpec((_TILE, D), lambda i: (i, 0)),
            pl.BlockSpec((_TILE, _M), lambda i: (i, 0)),
            pl.BlockSpec((1, 1), lambda i: (0, 0)),
        ),
        out_shape=(
            jax.ShapeDtypeStruct((bn, D), jnp.float32),
            jax.ShapeDtypeStruct((bn, _M), jnp.int32),
            jax.ShapeDtypeStruct((1, 1), jnp.float32),
        ),
        scratch_shapes=[
            pltpu.VMEM((_DB, _M * _P), jnp.float32),
            pltpu.VMEM((1, _M * _P), jnp.float32),
            pltpu.VMEM((_M * _P, _DB), jnp.bfloat16),
            pltpu.VMEM((_M * _P, _DB), jnp.bfloat16),
        ],
    )(flat_q, mem)
    embeddings = emb.reshape(B, N, D)
    indices = idx.reshape(B, N, _M)
    vq_loss = jnp.zeros((), jnp.float32)
    commitment_loss = loss.reshape(())
    return (embeddings, indices, vq_loss, commitment_loss)


# speedup vs baseline: 26.4385x; 1.0645x over previous
import functools

import jax
import jax.numpy as jnp
from jax.experimental import pallas as pl
from jax.experimental.pallas import tpu as pltpu

_P = 512
_M = 2
_DB = 256
_D = _M * _DB
_TILE = 784


def _vq_kernel(q_ref, mem_ref, emb_ref, idx_ref, loss_ref,
               wdt2_ref, mnorm_ref, mhi_ref, mlo_ref, *, inv_count):
    step = pl.program_id(0)
    nsteps = pl.num_programs(0)

    @pl.when(step == 0)
    def _init():
        m = mem_ref[...]
        wdt2 = jnp.swapaxes(m, 0, 1) * -2.0
        wdt2_ref[...] = wdt2
        mnorm_ref[...] = 0.25 * jnp.sum(wdt2 * wdt2, axis=0,
                                        keepdims=True)
        mhi = m.astype(jnp.bfloat16)
        mhi_ref[...] = mhi
        mlo_ref[...] = (m - mhi.astype(jnp.float32)).astype(jnp.bfloat16)
        loss_ref[...] = jnp.zeros((1, 1), jnp.float32)

    q = q_ref[...]
    rows = q.shape[0]
    iota = jax.lax.broadcasted_iota(jnp.int32, (rows, _P), 1)
    loss_acc = jnp.zeros((), jnp.float32)
    for i in range(_M):
        qi = q[:, i * _DB:(i + 1) * _DB]
        scores2 = jax.lax.dot_general(
            qi, wdt2_ref[:, i * _P:(i + 1) * _P],
            (((1,), (0,)), ((), ())),
            precision=jax.lax.Precision.HIGHEST,
            preferred_element_type=jnp.float32)
        dist = mnorm_ref[:, i * _P:(i + 1) * _P] + scores2
        rmin = jnp.min(dist, axis=1, keepdims=True)
        ismin = dist == rmin
        idx = jnp.min(jnp.where(ismin, iota, _P),
                      axis=1, keepdims=True)
        idx_ref[:, i:i + 1] = idx + i * _P
        onehot = jnp.logical_and(ismin, iota == idx).astype(jnp.bfloat16)
        gathered = jax.lax.dot_general(
            onehot, mhi_ref[i * _P:(i + 1) * _P, :],
            (((1,), (0,)), ((), ())),
            preferred_element_type=jnp.float32)
        gathered = gathered + jax.lax.dot_general(
            onehot, mlo_ref[i * _P:(i + 1) * _P, :],
            (((1,), (0,)), ((), ())),
            preferred_element_type=jnp.float32)
        emb = (gathered - qi) + qi
        emb_ref[:, i * _DB:(i + 1) * _DB] = emb
        loss_acc = loss_acc + jnp.sum((emb - qi) ** 2)
    loss_ref[...] += jnp.reshape(loss_acc, (1, 1))

    @pl.when(step == nsteps - 1)
    def _final():
        loss_ref[...] *= inv_count


def kernel(queries, mem):
    B, N, D = queries.shape
    bn = B * N
    flat_q = queries.reshape(bn, D)
    grid = (bn // _TILE,)
    emb, idx, loss = pl.pallas_call(
        functools.partial(_vq_kernel, inv_count=1.0 / float(bn * D)),
        grid=grid,
        in_specs=[
            pl.BlockSpec((_TILE, D), lambda i: (i, 0)),
            pl.BlockSpec((_M * _P, _DB), lambda i: (0, 0)),
        ],
        out_specs=(
            pl.BlockSpec((_TILE, D), lambda i: (i, 0)),
            pl.BlockSpec((_TILE, _M), lambda i: (i, 0)),
            pl.BlockSpec((1, 1), lambda i: (0, 0)),
        ),
        out_shape=(
            jax.ShapeDtypeStruct((bn, D), jnp.float32),
            jax.ShapeDtypeStruct((bn, _M), jnp.int32),
            jax.ShapeDtypeStruct((1, 1), jnp.float32),
        ),
        scratch_shapes=[
            pltpu.VMEM((_DB, _M * _P), jnp.float32),
            pltpu.VMEM((1, _M * _P), jnp.float32),
            pltpu.VMEM((_M * _P, _DB), jnp.bfloat16),
            pltpu.VMEM((_M * _P, _DB), jnp.bfloat16),
        ],
    )(flat_q, mem)
    embeddings = emb.reshape(B, N, D)
    indices = idx.reshape(B, N, _M)
    vq_loss = jnp.zeros((), jnp.float32)
    commitment_loss = loss.reshape(())
    return (embeddings, indices, vq_loss, commitment_loss)


# speedup vs baseline: 28.8769x; 1.0922x over previous
import functools

import jax
import jax.numpy as jnp
from jax.experimental import pallas as pl
from jax.experimental.pallas import tpu as pltpu

_P = 512
_M = 2
_DB = 256
_D = _M * _DB
_TILE = 784


def _vq_kernel(q_ref, mem_ref, emb_ref, idx_ref, loss_ref,
               wdh_ref, wdl_ref, mnorm_ref, mhi_ref, mlo_ref, *, inv_count):
    step = pl.program_id(0)
    nsteps = pl.num_programs(0)

    @pl.when(step == 0)
    def _init():
        m = mem_ref[...]
        wdt2 = jnp.swapaxes(m, 0, 1) * -2.0
        mnorm_ref[...] = 0.25 * jnp.sum(wdt2 * wdt2, axis=0,
                                        keepdims=True)
        wdh = wdt2.astype(jnp.bfloat16)
        wdh_ref[...] = wdh
        wdl_ref[...] = (wdt2 - wdh.astype(jnp.float32)).astype(jnp.bfloat16)
        mhi = m.astype(jnp.bfloat16)
        mhi_ref[...] = mhi
        mlo_ref[...] = (m - mhi.astype(jnp.float32)).astype(jnp.bfloat16)
        loss_ref[...] = jnp.zeros((1, 1), jnp.float32)

    q = q_ref[...]
    qh = q.astype(jnp.bfloat16)
    ql = (q - qh.astype(jnp.float32)).astype(jnp.bfloat16)
    rows = q.shape[0]
    iota = jax.lax.broadcasted_iota(jnp.int32, (rows, _P), 1)
    loss_acc = jnp.zeros((), jnp.float32)
    dn = (((1,), (0,)), ((), ()))
    for i in range(_M):
        qi = q[:, i * _DB:(i + 1) * _DB]
        qhi = qh[:, i * _DB:(i + 1) * _DB]
        qli = ql[:, i * _DB:(i + 1) * _DB]
        wdhi = wdh_ref[:, i * _P:(i + 1) * _P]
        wdli = wdl_ref[:, i * _P:(i + 1) * _P]
        scores2 = jax.lax.dot_general(
            qhi, wdhi, dn, preferred_element_type=jnp.float32)
        scores2 = scores2 + jax.lax.dot_general(
            qhi, wdli, dn, preferred_element_type=jnp.float32)
        scores2 = scores2 + jax.lax.dot_general(
            qli, wdhi, dn, preferred_element_type=jnp.float32)
        dist = mnorm_ref[:, i * _P:(i + 1) * _P] + scores2
        rmin = jnp.min(dist, axis=1, keepdims=True)
        ismin = dist == rmin
        idx = jnp.min(jnp.where(ismin, iota, _P),
                      axis=1, keepdims=True)
        idx_ref[:, i:i + 1] = idx + i * _P
        onehot = jnp.logical_and(ismin, iota == idx).astype(jnp.bfloat16)
        gathered = jax.lax.dot_general(
            onehot, mhi_ref[i * _P:(i + 1) * _P, :],
            dn, preferred_element_type=jnp.float32)
        gathered = gathered + jax.lax.dot_general(
            onehot, mlo_ref[i * _P:(i + 1) * _P, :],
            dn, preferred_element_type=jnp.float32)
        emb = (gathered - qi) + qi
        emb_ref[:, i * _DB:(i + 1) * _DB] = emb
        loss_acc = loss_acc + jnp.sum((emb - qi) ** 2)
    loss_ref[...] += jnp.reshape(loss_acc, (1, 1))

    @pl.when(step == nsteps - 1)
    def _final():
        loss_ref[...] *= inv_count


def kernel(queries, mem):
    B, N, D = queries.shape
    bn = B * N
    flat_q = queries.reshape(bn, D)
    grid = (bn // _TILE,)
    emb, idx, loss = pl.pallas_call(
        functools.partial(_vq_kernel, inv_count=1.0 / float(bn * D)),
        grid=grid,
        in_specs=[
            pl.BlockSpec((_TILE, D), lambda i: (i, 0)),
            pl.BlockSpec((_M * _P, _DB), lambda i: (0, 0)),
        ],
        out_specs=(
            pl.BlockSpec((_TILE, D), lambda i: (i, 0)),
            pl.BlockSpec((_TILE, _M), lambda i: (i, 0)),
            pl.BlockSpec((1, 1), lambda i: (0, 0)),
        ),
        out_shape=(
            jax.ShapeDtypeStruct((bn, D), jnp.float32),
            jax.ShapeDtypeStruct((bn, _M), jnp.int32),
            jax.ShapeDtypeStruct((1, 1), jnp.float32),
        ),
        scratch_shapes=[
            pltpu.VMEM((_DB, _M * _P), jnp.bfloat16),
            pltpu.VMEM((_DB, _M * _P), jnp.bfloat16),
            pltpu.VMEM((1, _M * _P), jnp.float32),
            pltpu.VMEM((_M * _P, _DB), jnp.bfloat16),
            pltpu.VMEM((_M * _P, _DB), jnp.bfloat16),
        ],
    )(flat_q, mem)
    embeddings = emb.reshape(B, N, D)
    indices = idx.reshape(B, N, _M)
    vq_loss = jnp.zeros((), jnp.float32)
    commitment_loss = loss.reshape(())
    return (embeddings, indices, vq_loss, commitment_loss)
